# Initial kernel scaffold; baseline (speedup 1.0000x reference)
import jax
import jax.numpy as jnp
from jax.experimental import pallas as pl


def kernel(x, edge_index, batch_index, W, b):
    raise NotImplementedError("write your pallas kernel here")


# speedup vs baseline: 9.7414x; 9.7414x over previous
import functools
import jax
import jax.numpy as jnp
from jax import lax
from jax.experimental import pallas as pl
from jax.experimental.pallas import tpu as pltpu, tpu_sc as plsc

N = 10000
E = 320000
D = 128

NC = 2
NS = 16
NW = NC * NS
K = 128

NPAD = 10240
EPAD = 327680
E_PER_SC = EPAD // NC
E_PER_TILE = E_PER_SC // NS
N_CHUNKS = E_PER_TILE // K
ZROWS = NPAD // NS
WROWS = 624
BPAD = 12288

_sc_mesh = plsc.VectorSubcoreMesh(core_axis_name="c", subcore_axis_name="s")


@functools.partial(
    pl.kernel,
    out_type=jax.ShapeDtypeStruct((NW * NPAD,), jnp.float32),
    mesh=_sc_mesh,
    compiler_params=pltpu.CompilerParams(needs_layout_passes=False),
    scratch_types=[
        pltpu.VMEM((K,), jnp.int32),
        pltpu.VMEM((NPAD,), jnp.float32),
        pltpu.SemaphoreType.DMA,
    ],
)
def _deg_kernel(dst_hbm, out_hbm, idx_v, hist_v, sem):
    c = lax.axis_index("c")
    s = lax.axis_index("s")
    wid = c * NS + s

    def zero(i, _):
        hist_v[pl.ds(i * 16, 16)] = jnp.zeros((16,), jnp.float32)
        return 0

    lax.fori_loop(0, NPAD // 16, zero, 0)
    base = wid * E_PER_TILE
    ones16 = jnp.ones((16,), jnp.float32)

    def body(i, _):
        pltpu.sync_copy(dst_hbm.at[pl.ds(base + i * K, K)], idx_v)
        for j in range(K // 16):
            idx = idx_v[pl.ds(j * 16, 16)]
            plsc.addupdate_scatter(hist_v, [idx], ones16)
        return 0

    lax.fori_loop(0, N_CHUNKS, body, 0)
    pltpu.sync_copy(hist_v, out_hbm.at[pl.ds(wid * NPAD, NPAD)])


@functools.partial(
    pl.kernel,
    out_type=jax.ShapeDtypeStruct((NC * N, D), jnp.float32),
    mesh=_sc_mesh,
    scratch_types=[
        pltpu.VMEM((K,), jnp.int32),
        pltpu.VMEM((K,), jnp.int32),
        pltpu.VMEM((K, D), jnp.float32),
        pltpu.VMEM_SHARED((NPAD, D), jnp.float32),
        pltpu.SemaphoreType.DMA,
    ],
)
def _agg_kernel(g_hbm, src_hbm, dst_hbm, zeros_hbm, out_hbm,
                src_v, dst_v, rows_v, acc_sh, sem):
    c = lax.axis_index("c")
    s = lax.axis_index("s")
    pltpu.sync_copy(zeros_hbm, acc_sh.at[pl.ds(s * ZROWS, ZROWS)])
    plsc.subcore_barrier()
    base = c * E_PER_SC + s * E_PER_TILE

    def body(i, _):
        pltpu.sync_copy(src_hbm.at[pl.ds(base + i * K, K)], src_v)
        pltpu.sync_copy(dst_hbm.at[pl.ds(base + i * K, K)], dst_v)
        pltpu.async_copy(g_hbm.at[src_v], rows_v, sem).wait()
        pltpu.sync_copy(rows_v, acc_sh.at[dst_v], add=True)
        return 0

    lax.fori_loop(0, N_CHUNKS, body, 0)
    plsc.subcore_barrier()
    pltpu.sync_copy(
        acc_sh.at[pl.ds(s * WROWS, WROWS)],
        out_hbm.at[pl.ds(c * N + s * WROWS, WROWS)],
    )

    @pl.when(s == NS - 1)
    def _tail():
        t = NS * WROWS
        pltpu.sync_copy(
            acc_sh.at[pl.ds(t, N - t)],
            out_hbm.at[pl.ds(c * N + t, N - t)],
        )


@functools.partial(
    pl.kernel,
    out_type=jax.ShapeDtypeStruct((BPAD, D), jnp.float32),
    mesh=_sc_mesh,
    scratch_types=[
        pltpu.VMEM((K,), jnp.int32),
        pltpu.VMEM((K, D), jnp.float32),
        pltpu.SemaphoreType.DMA,
    ],
)
def _bgather_kernel(z_hbm, bidx_hbm, out_hbm, idx_v, rows_v, sem):
    c = lax.axis_index("c")
    s = lax.axis_index("s")
    wid = c * NS + s
    n_chunks = BPAD // (NW * K)

    def body(j, _):
        off = wid * (n_chunks * K) + j * K
        pltpu.sync_copy(bidx_hbm.at[pl.ds(off, K)], idx_v)
        pltpu.async_copy(z_hbm.at[idx_v], rows_v, sem).wait()
        pltpu.sync_copy(rows_v, out_hbm.at[pl.ds(off, K)])
        return 0

    lax.fori_loop(0, n_chunks, body, 0)


def _dinv(deg32):
    deg = jnp.sum(deg32, axis=0)[:N] + 1.0
    return lax.rsqrt(deg)


def _mm_body(x_ref, w_ref, deg32_ref, g_ref):
    h = jnp.dot(x_ref[...], w_ref[...], preferred_element_type=jnp.float32)
    g_ref[...] = h * _dinv(deg32_ref[...])[:, None]


def _epilogue_body(s2_ref, g_ref, deg32_ref, b_ref, z_ref):
    dinv = _dinv(deg32_ref[...])
    y = dinv[:, None] * (s2_ref[0] + s2_ref[1] + g_ref[...]) + b_ref[...]
    y = jnp.maximum(y, 0.0)
    m = jnp.max(y, axis=1, keepdims=True)
    t = y - m
    z_ref[...] = t - jnp.log(jnp.sum(jnp.exp(t), axis=1, keepdims=True))


def kernel(x, edge_index, batch_index, W, b):
    src = jnp.pad(edge_index[0], (0, EPAD - E))
    dst = jnp.pad(edge_index[1], (0, EPAD - E), constant_values=N)
    bidx = jnp.pad(batch_index, (0, BPAD - N))

    zerosD = jnp.zeros((ZROWS, D), jnp.float32)

    deg32 = _deg_kernel(dst).reshape(NW, NPAD)

    g = pl.pallas_call(
        _mm_body,
        out_shape=jax.ShapeDtypeStruct((N, D), jnp.float32),
    )(x, W, deg32)

    s2 = _agg_kernel(g, src, dst, zerosD).reshape(NC, N, D)

    z = pl.pallas_call(
        _epilogue_body,
        out_shape=jax.ShapeDtypeStruct((N, D), jnp.float32),
    )(s2, g, deg32, b.reshape(1, D))

    outp = _bgather_kernel(z, bidx)
    return outp[:N]


# speedup vs baseline: 12.3341x; 1.2662x over previous
import functools
import jax
import jax.numpy as jnp
from jax import lax
from jax.experimental import pallas as pl
from jax.experimental.pallas import tpu as pltpu, tpu_sc as plsc

N = 10000
E = 320000
D = 128

NC = 2
NS = 16
NW = NC * NS
K = 128

NPAD = 10240
EPAD = 327680
E_PER_SC = EPAD // NC
E_PER_TILE = E_PER_SC // NS
N_CHUNKS = E_PER_TILE // K
ZROWS = NPAD // NS
WROWS = 624
BPAD = 12288

_sc_mesh = plsc.VectorSubcoreMesh(core_axis_name="c", subcore_axis_name="s")


@functools.partial(
    pl.kernel,
    out_type=jax.ShapeDtypeStruct((NW * NPAD,), jnp.float32),
    mesh=_sc_mesh,
    compiler_params=pltpu.CompilerParams(needs_layout_passes=False),
    scratch_types=[
        pltpu.VMEM((N_CHUNKS, K), jnp.int32),
        pltpu.VMEM((NPAD,), jnp.float32),
        pltpu.SemaphoreType.DMA,
    ],
)
def _deg_kernel(dst_hbm, out_hbm, idx_st, hist_v, sem):
    c = lax.axis_index("c")
    s = lax.axis_index("s")
    wid = c * NS + s

    def zero(i, _):
        hist_v[pl.ds(i * 16, 16)] = jnp.zeros((16,), jnp.float32)
        return 0

    lax.fori_loop(0, NPAD // 16, zero, 0)
    pltpu.sync_copy(dst_hbm.at[wid * 2], idx_st.at[pl.ds(0, N_CHUNKS // 2)])
    pltpu.sync_copy(dst_hbm.at[wid * 2 + 1],
                    idx_st.at[pl.ds(N_CHUNKS // 2, N_CHUNKS // 2)])
    ones16 = jnp.ones((16,), jnp.float32)

    def body(i, _):
        for j in range(K // 16):
            idx = idx_st[i, pl.ds(j * 16, 16)]
            plsc.addupdate_scatter(hist_v, [idx], ones16)
        return 0

    lax.fori_loop(0, N_CHUNKS, body, 0)
    pltpu.sync_copy(hist_v, out_hbm.at[pl.ds(wid * NPAD, NPAD)])


@functools.partial(
    pl.kernel,
    out_type=jax.ShapeDtypeStruct((NC * N, D), jnp.float32),
    mesh=_sc_mesh,
    scratch_types=[
        pltpu.VMEM((N_CHUNKS // 2, K), jnp.int32),
        pltpu.VMEM((N_CHUNKS // 2, K), jnp.int32),
        pltpu.VMEM((K, D), jnp.float32),
        pltpu.VMEM((K, D), jnp.float32),
        pltpu.VMEM_SHARED((NPAD, D), jnp.float32),
        pltpu.SemaphoreType.DMA,
        pltpu.SemaphoreType.DMA,
    ],
)
def _agg_kernel(g_hbm, src_hbm, dst_hbm, zeros_hbm, out_hbm,
                src_st, dst_st, buf0, buf1, acc_sh, sem0, sem1):
    c = lax.axis_index("c")
    s = lax.axis_index("s")
    wid = c * NS + s
    HALF = N_CHUNKS // 2
    pltpu.sync_copy(zeros_hbm, acc_sh.at[pl.ds(s * ZROWS, ZROWS)])
    plsc.subcore_barrier()

    def start_gather(i, buf, sem):
        pltpu.async_copy(g_hbm.at[src_st.at[i]], buf, sem)

    def wait_gather(buf, sem):
        pltpu.make_async_copy(g_hbm.at[pl.ds(0, K)], buf, sem).wait()

    for h in range(2):
        pltpu.sync_copy(src_hbm.at[wid * 2 + h], src_st)
        pltpu.sync_copy(dst_hbm.at[wid * 2 + h], dst_st)
        start_gather(0, buf0, sem0)

        def body(j, _):
            i = 2 * j
            start_gather(i + 1, buf1, sem1)
            wait_gather(buf0, sem0)
            pltpu.sync_copy(buf0, acc_sh.at[dst_st.at[i]], add=True)

            @pl.when(i + 2 < HALF)
            def _():
                start_gather(i + 2, buf0, sem0)

            wait_gather(buf1, sem1)
            pltpu.sync_copy(buf1, acc_sh.at[dst_st.at[i + 1]], add=True)
            return 0

        lax.fori_loop(0, HALF // 2, body, 0)
    plsc.subcore_barrier()
    pltpu.sync_copy(
        acc_sh.at[pl.ds(s * WROWS, WROWS)],
        out_hbm.at[pl.ds(c * N + s * WROWS, WROWS)],
    )

    @pl.when(s == NS - 1)
    def _tail():
        t = NS * WROWS
        pltpu.sync_copy(
            acc_sh.at[pl.ds(t, N - t)],
            out_hbm.at[pl.ds(c * N + t, N - t)],
        )


_BCH = BPAD // (NW * K)


@functools.partial(
    pl.kernel,
    out_type=jax.ShapeDtypeStruct((BPAD, D), jnp.float32),
    mesh=_sc_mesh,
    scratch_types=[
        pltpu.VMEM((_BCH, K), jnp.int32),
        [pltpu.VMEM((K, D), jnp.float32) for _ in range(_BCH)],
        [pltpu.SemaphoreType.DMA for _ in range(_BCH)],
    ],
)
def _bgather_kernel(z_hbm, bidx_hbm, out_hbm, idx_st, bufs, sems):
    c = lax.axis_index("c")
    s = lax.axis_index("s")
    wid = c * NS + s
    pltpu.sync_copy(bidx_hbm.at[wid], idx_st)
    for j in range(_BCH):
        pltpu.async_copy(z_hbm.at[idx_st.at[j]], bufs[j], sems[j])
    for j in range(_BCH):
        pltpu.make_async_copy(z_hbm.at[pl.ds(0, K)], bufs[j], sems[j]).wait()
        pltpu.sync_copy(bufs[j], out_hbm.at[pl.ds(wid * (_BCH * K) + j * K, K)])


def _dinv(deg32):
    deg = jnp.sum(deg32, axis=0)[:N] + 1.0
    return lax.rsqrt(deg)


def _mm_body(x_ref, w_ref, deg32_ref, g_ref):
    h = jnp.dot(x_ref[...], w_ref[...], preferred_element_type=jnp.float32)
    g_ref[...] = h * _dinv(deg32_ref[...])[:, None]


def _epilogue_body(s2_ref, g_ref, deg32_ref, b_ref, z_ref):
    dinv = _dinv(deg32_ref[...])
    y = dinv[:, None] * (s2_ref[0] + s2_ref[1] + g_ref[...]) + b_ref[...]
    y = jnp.maximum(y, 0.0)
    m = jnp.max(y, axis=1, keepdims=True)
    t = y - m
    z_ref[...] = t - jnp.log(jnp.sum(jnp.exp(t), axis=1, keepdims=True))


def kernel(x, edge_index, batch_index, W, b):
    src = jnp.pad(edge_index[0], (0, EPAD - E)).reshape(NW * 2, N_CHUNKS // 2, K)
    dst = jnp.pad(edge_index[1], (0, EPAD - E),
                  constant_values=N).reshape(NW * 2, N_CHUNKS // 2, K)
    bidx = jnp.pad(batch_index, (0, BPAD - N)).reshape(NW, _BCH, K)

    zerosD = jnp.zeros((ZROWS, D), jnp.float32)

    deg32 = _deg_kernel(dst).reshape(NW, NPAD)

    g = pl.pallas_call(
        _mm_body,
        out_shape=jax.ShapeDtypeStruct((N, D), jnp.float32),
    )(x, W, deg32)

    s2 = _agg_kernel(g, src, dst, zerosD).reshape(NC, N, D)

    z = pl.pallas_call(
        _epilogue_body,
        out_shape=jax.ShapeDtypeStruct((N, D), jnp.float32),
    )(s2, g, deg32, b.reshape(1, D))

    outp = _bgather_kernel(z, bidx)
    return outp[:N]
